# Initial kernel scaffold; baseline (speedup 1.0000x reference)
import jax
import jax.numpy as jnp
from jax.experimental import pallas as pl


def kernel(inputs, embeddings):
    raise NotImplementedError("write your pallas kernel here")


# speedup vs baseline: 1.1109x; 1.1109x over previous
import functools

import jax
import jax.numpy as jnp
from jax import lax
from jax.experimental import pallas as pl
from jax.experimental.pallas import tpu as pltpu
from jax.experimental.pallas import tpu_sc as plsc

FLAT_N = 32 * 576
EMB_K = 8192
EMB_D = 64
TOK_T = 256
NUM_TILES = FLAT_N // TOK_T
CHUNK_BOUNDS = (0, 4096, 8192)
LOSS_SCALE = 1.25 / (FLAT_N * EMB_D)


def _argmin_body(flat_ref, emb_ref, idx_ref, loss_ref):
    flat = flat_ref[...]
    x2 = jnp.sum(flat * flat, axis=1, keepdims=True)
    ones = jnp.ones((1, EMB_D), jnp.float32)

    minval = jnp.full((TOK_T, 1), jnp.inf, jnp.bfloat16)
    chosen = jnp.full((TOK_T, 1), jnp.inf, jnp.float32)
    minidx = jnp.zeros((TOK_T, 1), jnp.int32)
    for lo, hi in zip(CHUNK_BOUNDS[:-1], CHUNK_BOUNDS[1:]):
        width = hi - lo
        emb_c = emb_ref[pl.ds(lo, width), :]
        e2 = lax.dot_general(ones, emb_c * emb_c,
                             (((1,), (1,)), ((), ())),
                             preferred_element_type=jnp.float32)
        xe = lax.dot_general(flat, emb_c,
                             (((1,), (1,)), ((), ())),
                             preferred_element_type=jnp.float32)
        d = (e2 + x2) - 2.0 * xe
        mv = jnp.min(d, axis=1, keepdims=True)
        iota = lax.broadcasted_iota(jnp.int32, (TOK_T, width), 1) + lo
        mi = jnp.min(jnp.where(d == mv, iota, EMB_K), axis=1, keepdims=True)
        better = mv < minval.astype(jnp.float32)
        minidx = jnp.where(better, mi, minidx)
        chosen = jnp.where(better, mv, chosen)
        minval = jnp.where(better, mv.astype(jnp.bfloat16), minval)

    idx_ref[0, :, :] = minidx

    @pl.when(pl.program_id(0) == 0)
    def _():
        loss_ref[0, 0] = 0.0

    loss_ref[0, 0] += jnp.sum(chosen)

    @pl.when(pl.program_id(0) == NUM_TILES - 1)
    def _():
        loss_ref[0, 0] = loss_ref[0, 0] * LOSS_SCALE


def _argmin_call(flat, embeddings):
    return pl.pallas_call(
        _argmin_body,
        grid=(NUM_TILES,),
        in_specs=[
            pl.BlockSpec((TOK_T, EMB_D), lambda i: (i, 0)),
            pl.BlockSpec((EMB_K, EMB_D), lambda i: (0, 0)),
        ],
        out_specs=[
            pl.BlockSpec((1, TOK_T, 1), lambda i: (i, 0, 0)),
            pl.BlockSpec(memory_space=pltpu.SMEM, block_shape=(1, 1),
                         index_map=lambda i: (0, 0)),
        ],
        out_shape=[
            jax.ShapeDtypeStruct((NUM_TILES, TOK_T, 1), jnp.int32),
            jax.ShapeDtypeStruct((1, 1), jnp.float32),
        ],
        compiler_params=pltpu.CompilerParams(
            dimension_semantics=("arbitrary",),
        ),
    )(flat, embeddings)


_NC = 2
_NS = 16
_NW = _NC * _NS
_B_PER_W = FLAT_N // _NW
_IDX_W = 96
_N_CH = _B_PER_W // _IDX_W
_PAD_D = 128


def _gather_body(table_hbm, idx_hbm, out_hbm, idx_v, rows_v, sem):
    wid = lax.axis_index("s") * _NC + lax.axis_index("c")
    pltpu.sync_copy(idx_hbm.at[wid], idx_v)
    copies = [
        pltpu.async_copy(table_hbm.at[idx_v.at[j]],
                         rows_v.at[pl.ds(j * _IDX_W, _IDX_W)], sem)
        for j in range(_N_CH)
    ]
    for c in copies:
        c.wait()
    pltpu.sync_copy(rows_v, out_hbm.at[pl.ds(wid * _B_PER_W, _B_PER_W)])


def _gather_call(table_pad, idx2):
    mesh = plsc.VectorSubcoreMesh(
        core_axis_name="c", subcore_axis_name="s",
        num_cores=_NC, num_subcores=_NS)
    return pl.kernel(
        _gather_body,
        out_type=jax.ShapeDtypeStruct((FLAT_N, _PAD_D), jnp.float32),
        mesh=mesh,
        scratch_types=[
            pltpu.VMEM((_N_CH, _IDX_W), jnp.int32),
            pltpu.VMEM((_B_PER_W, _PAD_D), jnp.float32),
            pltpu.SemaphoreType.DMA,
        ],
    )(table_pad, idx2)


def kernel(inputs, embeddings):
    flat = inputs.reshape(FLAT_N, EMB_D)
    idx3, loss = _argmin_call(flat, embeddings)
    idx2 = idx3.reshape(_NW, _N_CH, _IDX_W)
    table_pad = jnp.pad(embeddings, ((0, 0), (0, _PAD_D - EMB_D)))
    quantized = _gather_call(table_pad, idx2)[:, :EMB_D].reshape(inputs.shape)
    quantized_st = inputs + (quantized - inputs)
    return loss[0, 0], quantized_st
